# baseline (speedup 1.0000x reference)
import jax
import jax.numpy as jnp
from jax import lax
from jax.experimental import pallas as pl
from jax.experimental.pallas import tpu as pltpu
from jax.experimental.pallas import tpu_sc as plsc

NC = 2
NS = 16
NW = NC * NS
BATCH = 16384
DIM = 32
B_PER_W = BATCH // NW
RING = 4
LANE = 16


def _gmf_body(ut, it, uidx, iidx, out,
              uidx_v, iidx_v, uslot, islot, upack, ipack, out_v, sem):
    wid = lax.axis_index("s") * NC + lax.axis_index("c")

    pltpu.sync_copy(uidx.at[wid], uidx_v)
    pltpu.sync_copy(iidx.at[wid], iidx_v)

    lane = lax.iota(jnp.int32, LANE)

    def idx_scalar(ref, k):
        g16 = pl.multiple_of((k >> 4) << 4, LANE)
        v16 = ref[pl.ds(g16, LANE)]
        sel = lane == (k & (LANE - 1))
        return jnp.sum(jnp.where(sel, v16, 0))

    def fire(k):
        slot = lax.rem(k, RING)
        uv = idx_scalar(uidx_v, k)
        iv = idx_scalar(iidx_v, k)
        ub = pl.multiple_of((uv >> 7) << 7, 128)
        ib = pl.multiple_of((iv >> 7) << 7, 128)
        pltpu.async_copy(ut.at[:, pl.ds(ub, 128)], uslot.at[slot],
                         sem.at[slot])
        pltpu.async_copy(it.at[:, pl.ds(ib, 128)], islot.at[slot],
                         sem.at[slot])

    def collect(k):
        slot = lax.rem(k, RING)
        pltpu.make_async_copy(ut.at[:, pl.ds(0, 128)],
                              uslot.at[slot], sem.at[slot]).wait()
        pltpu.make_async_copy(ut.at[:, pl.ds(0, 128)],
                              islot.at[slot], sem.at[slot]).wait()
        uc = jnp.full((LANE,), idx_scalar(uidx_v, k) & 127, jnp.int32)
        ic = jnp.full((LANE,), idx_scalar(iidx_v, k) & 127, jnp.int32)
        kk = jnp.full((LANE,), k, jnp.int32)
        for half in range(2):
            d16 = lane + half * LANE
            u16 = plsc.load_gather(uslot.at[slot], [d16, uc])
            i16 = plsc.load_gather(islot.at[slot], [d16, ic])
            plsc.store_scatter(upack, [d16, kk], u16)
            plsc.store_scatter(ipack, [d16, kk], i16)

    def fetch_loop(k, carry):
        @pl.when(k >= RING)
        def _():
            collect(k - RING)

        fire(k)
        return carry

    lax.fori_loop(0, B_PER_W, fetch_loop, 0)

    def tail_loop(k, carry):
        collect(k)
        return carry

    lax.fori_loop(B_PER_W - RING, B_PER_W, tail_loop, 0)

    def block(b, carry):
        base = b * LANE
        acc = jnp.zeros((LANE,), jnp.float32)
        for d in range(DIM):
            acc = acc + upack[d, pl.ds(base, LANE)] * ipack[d, pl.ds(base, LANE)]
        out_v[pl.ds(base, LANE)] = 1.0 / (1.0 + jnp.exp(-acc))
        return carry

    lax.fori_loop(0, B_PER_W // LANE, block, 0)

    pltpu.sync_copy(out_v, out.at[pl.ds(wid * B_PER_W, B_PER_W)])


def kernel(user, item, user_table, item_table):
    uidx = user.astype(jnp.int32).reshape(NW, B_PER_W)
    iidx = item.astype(jnp.int32).reshape(NW, B_PER_W)
    ut = user_table.T
    it = item_table.T

    mesh = plsc.VectorSubcoreMesh(
        core_axis_name="c", subcore_axis_name="s",
        num_cores=NC, num_subcores=NS)

    run = pl.kernel(
        _gmf_body,
        out_type=jax.ShapeDtypeStruct((BATCH,), jnp.float32),
        mesh=mesh,
        scratch_types=[
            pltpu.VMEM((B_PER_W,), jnp.int32),
            pltpu.VMEM((B_PER_W,), jnp.int32),
            pltpu.VMEM((RING, DIM, 128), jnp.float32),
            pltpu.VMEM((RING, DIM, 128), jnp.float32),
            pltpu.VMEM((DIM, B_PER_W), jnp.float32),
            pltpu.VMEM((DIM, B_PER_W), jnp.float32),
            pltpu.VMEM((B_PER_W,), jnp.float32),
            pltpu.SemaphoreType.DMA((RING,)),
        ],
        compiler_params=pltpu.CompilerParams(needs_layout_passes=False),
    )
    return run(ut, it, uidx, iidx)


# speedup vs baseline: 1.0834x; 1.0834x over previous
import jax
import jax.numpy as jnp
from jax import lax
from jax.experimental import pallas as pl
from jax.experimental.pallas import tpu as pltpu
from jax.experimental.pallas import tpu_sc as plsc

NC = 2
NS = 16
NW = NC * NS
BATCH = 16384
DIM = 32
B_PER_W = BATCH // NW
RING = 6
LANE = 16
NGROUP = B_PER_W // LANE


def _gmf_body(ut, it, uidx, iidx, out,
              uidx_v, iidx_v, ucol_v, icol_v, uslot, islot,
              upack, ipack, out_v, sem):
    wid = lax.axis_index("s") * NC + lax.axis_index("c")

    pltpu.sync_copy(uidx.at[wid], uidx_v)
    pltpu.sync_copy(iidx.at[wid], iidx_v)

    lane = lax.iota(jnp.int32, LANE)

    def fire(k, j, ubase16, ibase16):
        slot = lax.rem(k, RING)
        sel = lane == j
        ub = pl.multiple_of(jnp.sum(jnp.where(sel, ubase16, 0)), 128)
        ib = pl.multiple_of(jnp.sum(jnp.where(sel, ibase16, 0)), 128)
        pltpu.async_copy(ut.at[:, pl.ds(ub, 128)], uslot.at[slot],
                         sem.at[slot])
        pltpu.async_copy(it.at[:, pl.ds(ib, 128)], islot.at[slot],
                         sem.at[slot])

    def collect(k):
        slot = lax.rem(k, RING)
        pltpu.make_async_copy(ut.at[:, pl.ds(0, 128)],
                              uslot.at[slot], sem.at[slot]).wait()
        pltpu.make_async_copy(ut.at[:, pl.ds(0, 128)],
                              islot.at[slot], sem.at[slot]).wait()
        kk = jnp.full((LANE,), k, jnp.int32)
        uc = plsc.load_gather(ucol_v, [kk])
        ic = plsc.load_gather(icol_v, [kk])
        for half in range(2):
            d16 = lane + half * LANE
            u16 = plsc.load_gather(uslot.at[slot], [d16, uc])
            i16 = plsc.load_gather(islot.at[slot], [d16, ic])
            plsc.store_scatter(upack, [d16, kk], u16)
            plsc.store_scatter(ipack, [d16, kk], i16)

    def group_loop(g, carry):
        base = pl.multiple_of(g * LANE, LANE)
        u16 = uidx_v[pl.ds(base, LANE)]
        i16 = iidx_v[pl.ds(base, LANE)]
        ubase16 = (u16 >> 7) << 7
        ibase16 = (i16 >> 7) << 7
        ucol_v[pl.ds(base, LANE)] = u16 & 127
        icol_v[pl.ds(base, LANE)] = i16 & 127
        for j in range(LANE):
            k = base + j

            @pl.when(k >= RING)
            def _():
                collect(k - RING)

            fire(k, j, ubase16, ibase16)
        return carry

    lax.fori_loop(0, NGROUP, group_loop, 0)

    def tail_loop(k, carry):
        collect(k)
        return carry

    lax.fori_loop(B_PER_W - RING, B_PER_W, tail_loop, 0)

    def block(b, carry):
        base = b * LANE
        acc = jnp.zeros((LANE,), jnp.float32)
        for d in range(DIM):
            acc = acc + upack[d, pl.ds(base, LANE)] * ipack[d, pl.ds(base, LANE)]
        out_v[pl.ds(base, LANE)] = 1.0 / (1.0 + jnp.exp(-acc))
        return carry

    lax.fori_loop(0, B_PER_W // LANE, block, 0)

    pltpu.sync_copy(out_v, out.at[pl.ds(wid * B_PER_W, B_PER_W)])


def kernel(user, item, user_table, item_table):
    uidx = user.astype(jnp.int32).reshape(NW, B_PER_W)
    iidx = item.astype(jnp.int32).reshape(NW, B_PER_W)
    ut = user_table.T
    it = item_table.T

    mesh = plsc.VectorSubcoreMesh(
        core_axis_name="c", subcore_axis_name="s",
        num_cores=NC, num_subcores=NS)

    run = pl.kernel(
        _gmf_body,
        out_type=jax.ShapeDtypeStruct((BATCH,), jnp.float32),
        mesh=mesh,
        scratch_types=[
            pltpu.VMEM((B_PER_W,), jnp.int32),
            pltpu.VMEM((B_PER_W,), jnp.int32),
            pltpu.VMEM((B_PER_W,), jnp.int32),
            pltpu.VMEM((B_PER_W,), jnp.int32),
            pltpu.VMEM((RING, DIM, 128), jnp.float32),
            pltpu.VMEM((RING, DIM, 128), jnp.float32),
            pltpu.VMEM((DIM, B_PER_W), jnp.float32),
            pltpu.VMEM((DIM, B_PER_W), jnp.float32),
            pltpu.VMEM((B_PER_W,), jnp.float32),
            pltpu.SemaphoreType.DMA((RING,)),
        ],
        compiler_params=pltpu.CompilerParams(needs_layout_passes=False),
    )
    return run(ut, it, uidx, iidx)


# speedup vs baseline: 1.1602x; 1.0709x over previous
import jax
import jax.numpy as jnp
from jax import lax
from jax.experimental import pallas as pl
from jax.experimental.pallas import tpu as pltpu
from jax.experimental.pallas import tpu_sc as plsc

NC = 2
NS = 16
NW = NC * NS
BATCH = 16384
DIM = 32
B_PER_W = BATCH // NW
RING = 10
LANE = 16
NGROUP = B_PER_W // LANE


def _gmf_body(ut, it, uidx, iidx, out,
              uidx_v, iidx_v, ucol_v, icol_v, uslot, islot,
              upack, ipack, out_v, sem):
    wid = lax.axis_index("s") * NC + lax.axis_index("c")

    pltpu.sync_copy(uidx.at[wid], uidx_v)
    pltpu.sync_copy(iidx.at[wid], iidx_v)

    lane = lax.iota(jnp.int32, LANE)

    def fire(k, j, ubase16, ibase16):
        slot = lax.rem(k, RING)
        sel = lane == j
        ub = pl.multiple_of(jnp.sum(jnp.where(sel, ubase16, 0)), 128)
        ib = pl.multiple_of(jnp.sum(jnp.where(sel, ibase16, 0)), 128)
        pltpu.async_copy(ut.at[:, pl.ds(ub, 128)], uslot.at[slot],
                         sem.at[slot])
        pltpu.async_copy(it.at[:, pl.ds(ib, 128)], islot.at[slot],
                         sem.at[slot])

    def collect(k):
        slot = lax.rem(k, RING)
        pltpu.make_async_copy(ut.at[:, pl.ds(0, 128)],
                              uslot.at[slot], sem.at[slot]).wait()
        pltpu.make_async_copy(ut.at[:, pl.ds(0, 128)],
                              islot.at[slot], sem.at[slot]).wait()
        kk = jnp.full((LANE,), k, jnp.int32)
        uc = plsc.load_gather(ucol_v, [kk])
        ic = plsc.load_gather(icol_v, [kk])
        for half in range(2):
            d16 = lane + half * LANE
            u16 = plsc.load_gather(uslot.at[slot], [d16, uc])
            i16 = plsc.load_gather(islot.at[slot], [d16, ic])
            plsc.store_scatter(upack, [d16, kk], u16)
            plsc.store_scatter(ipack, [d16, kk], i16)

    def group_loop(g, carry):
        base = pl.multiple_of(g * LANE, LANE)
        u16 = uidx_v[pl.ds(base, LANE)]
        i16 = iidx_v[pl.ds(base, LANE)]
        ubase16 = (u16 >> 7) << 7
        ibase16 = (i16 >> 7) << 7
        ucol_v[pl.ds(base, LANE)] = u16 & 127
        icol_v[pl.ds(base, LANE)] = i16 & 127
        for j in range(LANE):
            k = base + j

            @pl.when(k >= RING)
            def _():
                collect(k - RING)

            fire(k, j, ubase16, ibase16)
        return carry

    lax.fori_loop(0, NGROUP, group_loop, 0)

    def tail_loop(k, carry):
        collect(k)
        return carry

    lax.fori_loop(B_PER_W - RING, B_PER_W, tail_loop, 0)

    def block(b, carry):
        base = b * LANE
        acc = jnp.zeros((LANE,), jnp.float32)
        for d in range(DIM):
            acc = acc + upack[d, pl.ds(base, LANE)] * ipack[d, pl.ds(base, LANE)]
        out_v[pl.ds(base, LANE)] = 1.0 / (1.0 + jnp.exp(-acc))
        return carry

    lax.fori_loop(0, B_PER_W // LANE, block, 0)

    pltpu.sync_copy(out_v, out.at[pl.ds(wid * B_PER_W, B_PER_W)])


def kernel(user, item, user_table, item_table):
    uidx = user.astype(jnp.int32).reshape(NW, B_PER_W)
    iidx = item.astype(jnp.int32).reshape(NW, B_PER_W)
    ut = user_table.T
    it = item_table.T

    mesh = plsc.VectorSubcoreMesh(
        core_axis_name="c", subcore_axis_name="s",
        num_cores=NC, num_subcores=NS)

    run = pl.kernel(
        _gmf_body,
        out_type=jax.ShapeDtypeStruct((BATCH,), jnp.float32),
        mesh=mesh,
        scratch_types=[
            pltpu.VMEM((B_PER_W,), jnp.int32),
            pltpu.VMEM((B_PER_W,), jnp.int32),
            pltpu.VMEM((B_PER_W,), jnp.int32),
            pltpu.VMEM((B_PER_W,), jnp.int32),
            pltpu.VMEM((RING, DIM, 128), jnp.float32),
            pltpu.VMEM((RING, DIM, 128), jnp.float32),
            pltpu.VMEM((DIM, B_PER_W), jnp.float32),
            pltpu.VMEM((DIM, B_PER_W), jnp.float32),
            pltpu.VMEM((B_PER_W,), jnp.float32),
            pltpu.SemaphoreType.DMA((RING,)),
        ],
        compiler_params=pltpu.CompilerParams(needs_layout_passes=False),
    )
    return run(ut, it, uidx, iidx)


# speedup vs baseline: 1.1658x; 1.0048x over previous
import jax
import jax.numpy as jnp
from jax import lax
from jax.experimental import pallas as pl
from jax.experimental.pallas import tpu as pltpu
from jax.experimental.pallas import tpu_sc as plsc

NC = 2
NS = 16
NW = NC * NS
BATCH = 16384
DIM = 32
B_PER_W = BATCH // NW
RING = 11
LANE = 16
NGROUP = B_PER_W // LANE


def _gmf_body(ut, it, uidx, iidx, out,
              uidx_v, iidx_v, ucol_v, icol_v, uslot, islot,
              upack, ipack, out_v, sem):
    wid = lax.axis_index("s") * NC + lax.axis_index("c")

    pltpu.sync_copy(uidx.at[wid], uidx_v)
    pltpu.sync_copy(iidx.at[wid], iidx_v)

    lane = lax.iota(jnp.int32, LANE)

    def fire(k, j, ubase16, ibase16):
        slot = lax.rem(k, RING)
        sel = lane == j
        ub = pl.multiple_of(jnp.sum(jnp.where(sel, ubase16, 0)), 128)
        ib = pl.multiple_of(jnp.sum(jnp.where(sel, ibase16, 0)), 128)
        pltpu.async_copy(ut.at[:, pl.ds(ub, 128)], uslot.at[slot],
                         sem.at[slot])
        pltpu.async_copy(it.at[:, pl.ds(ib, 128)], islot.at[slot],
                         sem.at[slot])

    def collect(k):
        slot = lax.rem(k, RING)
        pltpu.make_async_copy(ut.at[:, pl.ds(0, 128)],
                              uslot.at[slot], sem.at[slot]).wait()
        pltpu.make_async_copy(ut.at[:, pl.ds(0, 128)],
                              islot.at[slot], sem.at[slot]).wait()
        kk = jnp.full((LANE,), k, jnp.int32)
        uc = plsc.load_gather(ucol_v, [kk])
        ic = plsc.load_gather(icol_v, [kk])
        for half in range(2):
            d16 = lane + half * LANE
            u16 = plsc.load_gather(uslot.at[slot], [d16, uc])
            i16 = plsc.load_gather(islot.at[slot], [d16, ic])
            plsc.store_scatter(upack, [d16, kk], u16)
            plsc.store_scatter(ipack, [d16, kk], i16)

    def group_loop(g, carry):
        base = pl.multiple_of(g * LANE, LANE)
        u16 = uidx_v[pl.ds(base, LANE)]
        i16 = iidx_v[pl.ds(base, LANE)]
        ubase16 = (u16 >> 7) << 7
        ibase16 = (i16 >> 7) << 7
        ucol_v[pl.ds(base, LANE)] = u16 & 127
        icol_v[pl.ds(base, LANE)] = i16 & 127
        for j in range(LANE):
            k = base + j

            @pl.when(k >= RING)
            def _():
                collect(k - RING)

            fire(k, j, ubase16, ibase16)
        return carry

    lax.fori_loop(0, NGROUP, group_loop, 0)

    def tail_loop(k, carry):
        collect(k)
        return carry

    lax.fori_loop(B_PER_W - RING, B_PER_W, tail_loop, 0)

    def block(b, carry):
        base = b * LANE
        acc = jnp.zeros((LANE,), jnp.float32)
        for d in range(DIM):
            acc = acc + upack[d, pl.ds(base, LANE)] * ipack[d, pl.ds(base, LANE)]
        out_v[pl.ds(base, LANE)] = 1.0 / (1.0 + jnp.exp(-acc))
        return carry

    lax.fori_loop(0, B_PER_W // LANE, block, 0)

    pltpu.sync_copy(out_v, out.at[pl.ds(wid * B_PER_W, B_PER_W)])


def kernel(user, item, user_table, item_table):
    uidx = user.astype(jnp.int32).reshape(NW, B_PER_W)
    iidx = item.astype(jnp.int32).reshape(NW, B_PER_W)
    ut = user_table.T
    it = item_table.T

    mesh = plsc.VectorSubcoreMesh(
        core_axis_name="c", subcore_axis_name="s",
        num_cores=NC, num_subcores=NS)

    run = pl.kernel(
        _gmf_body,
        out_type=jax.ShapeDtypeStruct((BATCH,), jnp.float32),
        mesh=mesh,
        scratch_types=[
            pltpu.VMEM((B_PER_W,), jnp.int32),
            pltpu.VMEM((B_PER_W,), jnp.int32),
            pltpu.VMEM((B_PER_W,), jnp.int32),
            pltpu.VMEM((B_PER_W,), jnp.int32),
            pltpu.VMEM((RING, DIM, 128), jnp.float32),
            pltpu.VMEM((RING, DIM, 128), jnp.float32),
            pltpu.VMEM((DIM, B_PER_W), jnp.float32),
            pltpu.VMEM((DIM, B_PER_W), jnp.float32),
            pltpu.VMEM((B_PER_W,), jnp.float32),
            pltpu.SemaphoreType.DMA((RING,)),
        ],
        compiler_params=pltpu.CompilerParams(needs_layout_passes=False),
    )
    return run(ut, it, uidx, iidx)
